# baseline (speedup 1.0000x reference)
import functools

import jax
import jax.numpy as jnp
from jax import lax
from jax.experimental import pallas as pl
from jax.experimental.pallas import tpu as pltpu
from jax.experimental.pallas import tpu_sc as plsc

VOCAB = 1000
EMBED = 32
BATCH = 4096
HIST = 200

LANE = 128
ROWS = BATCH * HIST // LANE
NWORKERS = 32
RPW = ROWS // NWORKERS
CH = 4
NCHUNK = RPW // CH
NS = 4

_mesh = plsc.VectorSubcoreMesh(core_axis_name="c", subcore_axis_name="s")


@functools.partial(
    pl.kernel,
    mesh=_mesh,
    out_type=jax.ShapeDtypeStruct((ROWS, LANE, EMBED), jnp.float32),
    scratch_types=[
        pltpu.VMEM((RPW, LANE), jnp.int32),
        pltpu.VMEM((NS, CH, LANE, EMBED), jnp.float32),
        pltpu.SemaphoreType.DMA((NS,)),
        pltpu.SemaphoreType.DMA((NS,)),
    ],
    compiler_params=pltpu.CompilerParams(use_tc_tiling_on_sc=False),
)
def _gather_kernel(idx_hbm, table_hbm, out_hbm, idx_v, rows_v, gat_sems, out_sems):
    wid = lax.axis_index("s") * 2 + lax.axis_index("c")
    base = wid * RPW
    pltpu.sync_copy(idx_hbm.at[pl.ds(base, RPW)], idx_v)

    def fire_gathers(j):
        s = j % NS
        return [
            pltpu.async_copy(
                table_hbm.at[idx_v.at[j * CH + k]],
                rows_v.at[s].at[k],
                gat_sems.at[s],
            )
            for k in range(CH)
        ]

    out_handles = [None] * NCHUNK
    gat_handles = fire_gathers(0)
    for j in range(NCHUNK):
        if j + 1 < NCHUNK:
            if j + 1 >= NS:
                out_handles[j + 1 - NS].wait()
            next_handles = fire_gathers(j + 1)
        else:
            next_handles = None
        for h in gat_handles:
            h.wait()
        out_handles[j] = pltpu.async_copy(
            rows_v.at[j % NS],
            out_hbm.at[pl.ds(base + j * CH, CH)],
            out_sems.at[j % NS],
        )
        gat_handles = next_handles
    for j in range(NCHUNK - NS, NCHUNK):
        out_handles[j].wait()


def kernel(word_idx_list, W):
    idx = word_idx_list.astype(jnp.int32).reshape(ROWS, LANE)
    out = _gather_kernel(idx, W)
    return out.reshape(BATCH, HIST, EMBED)


# speedup vs baseline: 1.2928x; 1.2928x over previous
import functools

import jax
import jax.numpy as jnp
from jax import lax
from jax.experimental import pallas as pl
from jax.experimental.pallas import tpu as pltpu
from jax.experimental.pallas import tpu_sc as plsc

VOCAB = 1000
EMBED = 32
BATCH = 4096
HIST = 200

LANE = 128
ROWS = BATCH * HIST // LANE
NWORKERS = 32
RPW = ROWS // NWORKERS
CH = 4
NCHUNK = RPW // CH
NS = 4

_mesh = plsc.VectorSubcoreMesh(core_axis_name="c", subcore_axis_name="s")


@functools.partial(
    pl.kernel,
    mesh=_mesh,
    out_type=jax.ShapeDtypeStruct((ROWS, LANE, EMBED), jnp.float32),
    scratch_types=[
        pltpu.VMEM((RPW, LANE), jnp.int32),
        pltpu.VMEM((NS, CH, LANE, EMBED), jnp.float32),
        pltpu.VMEM_SHARED((VOCAB, EMBED), jnp.float32),
        pltpu.SemaphoreType.DMA((NS,)),
        pltpu.SemaphoreType.DMA((NS,)),
    ],
    compiler_params=pltpu.CompilerParams(use_tc_tiling_on_sc=False),
)
def _gather_kernel(idx_hbm, table_hbm, out_hbm, idx_v, rows_v, table_sh,
                   gat_sems, out_sems):
    sid = lax.axis_index("s")
    wid = sid * 2 + lax.axis_index("c")
    base = wid * RPW

    @pl.when(sid == 0)
    def _stage_table():
        pltpu.sync_copy(table_hbm, table_sh)

    pltpu.sync_copy(idx_hbm.at[pl.ds(base, RPW)], idx_v)
    plsc.subcore_barrier()

    def fire_gathers(j):
        s = j % NS
        return [
            pltpu.async_copy(
                table_sh.at[idx_v.at[j * CH + k]],
                rows_v.at[s].at[k],
                gat_sems.at[s],
            )
            for k in range(CH)
        ]

    out_handles = [None] * NCHUNK
    gat_handles = fire_gathers(0)
    for j in range(NCHUNK):
        if j + 1 < NCHUNK:
            if j + 1 >= NS:
                out_handles[j + 1 - NS].wait()
            next_handles = fire_gathers(j + 1)
        else:
            next_handles = None
        for h in gat_handles:
            h.wait()
        out_handles[j] = pltpu.async_copy(
            rows_v.at[j % NS],
            out_hbm.at[pl.ds(base + j * CH, CH)],
            out_sems.at[j % NS],
        )
        gat_handles = next_handles
    for j in range(NCHUNK - NS, NCHUNK):
        out_handles[j].wait()


def kernel(word_idx_list, W):
    idx = word_idx_list.astype(jnp.int32).reshape(ROWS, LANE)
    out = _gather_kernel(idx, W)
    return out.reshape(BATCH, HIST, EMBED)
